# baseline (speedup 1.0000x reference)
import functools

import jax
import jax.numpy as jnp
from jax import lax
from jax.experimental import pallas as pl
from jax.experimental.pallas import tpu as pltpu
from jax.experimental.pallas import tpu_sc as plsc

N = 10000
D = 128
NB = 8
K = 16
RAD2 = (0.02 * 2.1) ** 2

NW = 32
NPAD = 10240
QPW = NPAD // NW
GQ = 16
EPAD = NPAD * K
EPW = EPAD // NW
GE = 128
CH = 64
QB = 64
EB = QB * K
NEG_INF = float("-inf")


def _leaky(z):
    return jnp.where(z >= 0, z, 0.01 * z)


def _sweep_body(posx_h, posy_h, posz_h, qlo_h, qhi_h, t_h, g_h, cnt_h,
                px, py, pz, qlo_v, qhi_v, gbuf0, gbuf1, cbuf, nbuf, cq,
                rows0, rows1, gsem0, gsem1, wsem0, wsem1):
    wid = lax.axis_index("s") * 2 + lax.axis_index("c")
    base = wid * QPW
    pltpu.sync_copy(posx_h, px)
    pltpu.sync_copy(posy_h, py)
    pltpu.sync_copy(posz_h, pz)
    pltpu.sync_copy(qlo_h.at[pl.ds(base, QPW)], qlo_v)
    pltpu.sync_copy(qhi_h.at[pl.ds(base, QPW)], qhi_v)

    lanes = lax.broadcasted_iota(jnp.int32, (16,), 0)
    zeros16 = jnp.zeros((16,), jnp.int32)
    ones16 = jnp.ones((16,), jnp.int32)
    c31 = jnp.full((16,), 31, jnp.int32)
    k16 = jnp.full((16,), K, jnp.int32)
    slots = ((gbuf0, rows0, gsem0, wsem0), (gbuf1, rows1, gsem1, wsem1))
    NG = QPW // GQ
    GEB = GQ * K

    def scan_group(g, gbuf):
        goff = base + g * GQ
        qlo_g = qlo_v[pl.ds(g * GQ, 16)]
        qhi_g = qhi_v[pl.ds(g * GQ, 16)]
        qpx_g = px[pl.ds(goff, 16)]
        qpy_g = py[pl.ds(goff, 16)]
        qpz_g = pz[pl.ds(goff, 16)]
        for q in range(GQ):
            lo = qlo_g[q]
            hi = qhi_g[q]
            qx = qpx_g[q]
            qy = qpy_g[q]
            qz = qpz_g[q]
            nbuf[pl.ds(0, 16)] = zeros16
            nbuf[pl.ds(16, 16)] = zeros16
            cq[...] = zeros16
            ptr0 = (lo // 16) * 16
            nch = (hi - ptr0 + (CH - 1)) // CH

            def chunk(ci, carry):
                cnt0 = cq[...][0]

                @pl.when(cnt0 < K)
                def _():
                    cnt = cnt0
                    for u in range(CH // 16):
                        ptr = ptr0 + ci * CH + u * 16
                        ids = ptr + lanes
                        dx = px[pl.ds(ptr, 16)] - qx
                        dy = py[pl.ds(ptr, 16)] - qy
                        dz = pz[pl.ds(ptr, 16)] - qz
                        d2 = dx * dx + dy * dy + dz * dz
                        val = (d2 <= RAD2) & (ids >= lo) & (ids < hi)
                        pcs = plsc.cumsum(jnp.where(val, ones16, zeros16))
                        pos = jnp.where(val, jnp.minimum(cnt + pcs - 1, c31),
                                        c31)
                        plsc.store_scatter(nbuf, [pos], ids)
                        cnt = cnt + pcs[15]
                    cq[...] = zeros16 + cnt

                return carry

            lax.fori_loop(0, nch, chunk, 0)
            cnt_f = cq[...][0]
            nb = nbuf[pl.ds(0, 16)]
            gbuf[pl.ds(q * 16, 16)] = jnp.where(lanes < cnt_f, nb,
                                                zeros16 + nb[0])
            cbuf[...] = jnp.where(lanes == q,
                                  jnp.minimum(zeros16 + cnt_f, k16), cbuf[...])
        pltpu.sync_copy(cbuf, cnt_h.at[pl.ds(base + g * GQ, GQ)])

    def pair_body(g2, carry):
        for p in range(2):
            gbuf, rows, gsem, wsem = slots[p]
            g = g2 * 2 + p
            ebase = (base + g * GQ) * K
            eprev = ebase - 2 * GEB

            @pl.when(g2 >= 1)
            def _fire_writeback():
                pltpu.make_async_copy(g_h.at[pl.ds(0, GEB)], rows,
                                      gsem).wait()
                pltpu.async_copy(rows, g_h.at[pl.ds(eprev, GEB)], wsem)

            scan_group(g, gbuf)

            @pl.when(g2 >= 1)
            def _wait_writeback():
                pltpu.make_async_copy(rows, g_h.at[pl.ds(eprev, GEB)],
                                      wsem).wait()

            pltpu.async_copy(t_h.at[gbuf.at[pl.ds(0, GE)]],
                             rows.at[pl.ds(0, GE)], gsem)
            pltpu.async_copy(t_h.at[gbuf.at[pl.ds(GE, GE)]],
                             rows.at[pl.ds(GE, GE)], gsem)
        return carry

    lax.fori_loop(0, NG // 2, pair_body, 0)
    for p in range(2):
        gbuf, rows, gsem, wsem = slots[p]
        g = NG - 2 + p
        pltpu.make_async_copy(g_h.at[pl.ds(0, GEB)], rows, gsem).wait()
        pltpu.sync_copy(rows, g_h.at[pl.ds((base + g * GQ) * K, GEB)])


@functools.cache
def _sweep_kernel():
    return functools.partial(
        pl.kernel,
        mesh=plsc.VectorSubcoreMesh(core_axis_name="c", subcore_axis_name="s"),
        compiler_params=pltpu.CompilerParams(needs_layout_passes=False),
        out_type=[jax.ShapeDtypeStruct((EPAD, D), jnp.float32),
                  jax.ShapeDtypeStruct((NPAD,), jnp.int32)],
        scratch_types=[pltpu.VMEM((NPAD,), jnp.float32),
                       pltpu.VMEM((NPAD,), jnp.float32),
                       pltpu.VMEM((NPAD,), jnp.float32),
                       pltpu.VMEM((QPW,), jnp.int32),
                       pltpu.VMEM((QPW,), jnp.int32),
                       pltpu.VMEM((GQ * K,), jnp.int32),
                       pltpu.VMEM((GQ * K,), jnp.int32),
                       pltpu.VMEM((GQ,), jnp.int32),
                       pltpu.VMEM((32,), jnp.int32),
                       pltpu.VMEM((16,), jnp.int32),
                       pltpu.VMEM((GQ * K, D), jnp.float32),
                       pltpu.VMEM((GQ * K, D), jnp.float32),
                       pltpu.SemaphoreType.DMA,
                       pltpu.SemaphoreType.DMA,
                       pltpu.SemaphoreType.DMA,
                       pltpu.SemaphoreType.DMA],
    )(_sweep_body)


def _sweep_call(*args):
    return _sweep_kernel()(*args)


def _tv_body(x_ref, p_ref, w1xt_ref, w1pt_ref, b1_ref, t_ref, v_ref):
    v = jnp.dot(p_ref[...], w1pt_ref[...], preferred_element_type=jnp.float32,
                precision=lax.Precision.HIGHEST)
    t = jnp.dot(x_ref[...], w1xt_ref[...], preferred_element_type=jnp.float32,
                precision=lax.Precision.HIGHEST)
    v_ref[...] = v
    t_ref[...] = t + v + b1_ref[0:1, :]


def _z1_block(g_ref, v_ref):
    vexp = jnp.broadcast_to(v_ref[...].reshape(QB, 1, D), (QB, K, D))
    return _leaky(g_ref[...] - vexp.reshape(QB * K, D))


def _stats1_body(g_ref, v_ref, fm_ref, s_ref):
    z1 = _z1_block(g_ref, v_ref)
    mrow = fm_ref[0]

    @pl.when(pl.program_id(0) == 0)
    def _():
        s_ref[...] = jnp.zeros_like(s_ref)

    s_ref[0:1, :] += jnp.dot(mrow, z1, preferred_element_type=jnp.float32,
                             precision=lax.Precision.HIGHEST)
    s_ref[1:2, :] += jnp.dot(mrow, z1 * z1,
                             preferred_element_type=jnp.float32,
                             precision=lax.Precision.HIGHEST)


def _layer2_body(g_ref, v_ref, fm_ref, w2t_ref, b2_ref, mx_ref, s_ref):
    z1 = _z1_block(g_ref, v_ref)
    z2 = jnp.dot(z1, w2t_ref[...], preferred_element_type=jnp.float32,
                 precision=lax.Precision.HIGHEST)
    z2 = _leaky(z2 + b2_ref[0:1, :])
    mrow = fm_ref[0]

    @pl.when(pl.program_id(0) == 0)
    def _():
        s_ref[...] = jnp.zeros_like(s_ref)

    s_ref[0:1, :] += jnp.dot(mrow, z2, preferred_element_type=jnp.float32,
                             precision=lax.Precision.HIGHEST)
    s_ref[1:2, :] += jnp.dot(mrow, z2 * z2,
                             preferred_element_type=jnp.float32,
                             precision=lax.Precision.HIGHEST)
    mx_ref[...] = jnp.max(z2.reshape(QB, K, D), axis=1)


def _final_body(m_ref, prm_ref, o_ref):
    o_ref[...] = m_ref[...] * prm_ref[0:1, :] + prm_ref[1:2, :]


def kernel(x, pos, batch, reflectance, sf, W1, b1, g1, be1, W2, b2, g2, be2):
    f32 = jnp.float32
    pos_s = pos / sf[batch][:, None]
    ar = jnp.arange(NB, dtype=batch.dtype)
    seg_lo = jnp.searchsorted(batch, ar, side="left").astype(jnp.int32)
    seg_hi = jnp.searchsorted(batch, ar, side="right").astype(jnp.int32)
    qlo = jnp.zeros((NPAD,), jnp.int32).at[:N].set(seg_lo[batch])
    qhi = jnp.zeros((NPAD,), jnp.int32).at[:N].set(seg_hi[batch])
    posx = jnp.zeros((NPAD,), f32).at[:N].set(pos[:, 0])
    posy = jnp.zeros((NPAD,), f32).at[:N].set(pos[:, 1])
    posz = jnp.zeros((NPAD,), f32).at[:N].set(pos[:, 2])
    xp = jnp.zeros((NPAD, D), f32).at[:N].set(x)
    pp = jnp.zeros((NPAD, D), f32).at[:N, :3].set(pos_s)
    w1xt = W1[:, :D].T
    w1pt = jnp.zeros((D, D), f32).at[:3, :].set(W1[:, D:].T)
    b1r = jnp.broadcast_to(b1.reshape(1, D), (8, D))

    nb_tv = NPAD // 512
    t, v = pl.pallas_call(
        _tv_body,
        grid=(nb_tv,),
        in_specs=[pl.BlockSpec((512, D), lambda i: (i, 0)),
                  pl.BlockSpec((512, D), lambda i: (i, 0)),
                  pl.BlockSpec((D, D), lambda i: (0, 0)),
                  pl.BlockSpec((D, D), lambda i: (0, 0)),
                  pl.BlockSpec((8, D), lambda i: (0, 0))],
        out_specs=[pl.BlockSpec((512, D), lambda i: (i, 0)),
                   pl.BlockSpec((512, D), lambda i: (i, 0))],
        out_shape=[jax.ShapeDtypeStruct((NPAD, D), f32),
                   jax.ShapeDtypeStruct((NPAD, D), f32)],
    )(xp, pp, w1xt, w1pt, b1r)

    G, counts = _sweep_call(posx, posy, posz, qlo, qhi, t)

    fmask = (jnp.arange(K, dtype=jnp.int32)[None, :]
             < counts[:, None]).astype(f32)
    nblk = NPAD // QB
    fmask3 = fmask.reshape(nblk, 1, EB)
    edge_specs = [pl.BlockSpec((EB, D), lambda i: (i, 0)),
                  pl.BlockSpec((QB, D), lambda i: (i, 0)),
                  pl.BlockSpec((1, 1, EB), lambda i: (i, 0, 0))]

    s1 = pl.pallas_call(
        _stats1_body,
        grid=(nblk,),
        in_specs=edge_specs,
        out_specs=pl.BlockSpec((8, D), lambda i: (0, 0)),
        out_shape=jax.ShapeDtypeStruct((8, D), f32),
    )(G, v, fmask3)

    cnt_tot = jnp.maximum(jnp.sum(counts.astype(f32)), 1.0)
    mu1 = s1[0] / cnt_tot
    var1 = jnp.maximum(s1[1] / cnt_tot - mu1 * mu1, 0.0)
    a1 = g1 * lax.rsqrt(var1 + 1e-5)
    w2t = (W2 * a1[None, :]).T
    b2f = b2 + W2 @ (be1 - mu1 * a1)
    b2r = jnp.broadcast_to(b2f.reshape(1, D), (8, D))

    m, s2 = pl.pallas_call(
        _layer2_body,
        grid=(nblk,),
        in_specs=edge_specs + [pl.BlockSpec((D, D), lambda i: (0, 0)),
                               pl.BlockSpec((8, D), lambda i: (0, 0))],
        out_specs=[pl.BlockSpec((QB, D), lambda i: (i, 0)),
                   pl.BlockSpec((8, D), lambda i: (0, 0))],
        out_shape=[jax.ShapeDtypeStruct((NPAD, D), f32),
                   jax.ShapeDtypeStruct((8, D), f32)],
    )(G, v, fmask3, w2t, b2r)

    mu2 = s2[0] / cnt_tot
    var2 = jnp.maximum(s2[1] / cnt_tot - mu2 * mu2, 0.0)
    scale2 = g2 * lax.rsqrt(var2 + 1e-5)
    shift2 = be2 - mu2 * scale2
    prm = jnp.zeros((8, D), f32).at[0].set(scale2).at[1].set(shift2)

    out_p = pl.pallas_call(
        _final_body,
        grid=(NPAD // 512,),
        in_specs=[pl.BlockSpec((512, D), lambda i: (i, 0)),
                  pl.BlockSpec((8, D), lambda i: (0, 0))],
        out_specs=pl.BlockSpec((512, D), lambda i: (i, 0)),
        out_shape=jax.ShapeDtypeStruct((NPAD, D), f32),
    )(m, prm)

    return (out_p[:N], pos, batch, reflectance, sf)


# speedup vs baseline: 1.2185x; 1.2185x over previous
import functools

import jax
import jax.numpy as jnp
from jax import lax
from jax.experimental import pallas as pl
from jax.experimental.pallas import tpu as pltpu
from jax.experimental.pallas import tpu_sc as plsc

N = 10000
D = 128
NB = 8
K = 16
RAD2 = (0.02 * 2.1) ** 2

NW = 32
NPAD = 10240
QPW = NPAD // NW
GQ = 16
EPAD = NPAD * K
EPW = EPAD // NW
GE = 128
CH = 64
QB = 64
EB = QB * K
NEG_INF = float("-inf")


def _leaky(z):
    return jnp.where(z >= 0, z, 0.01 * z)


def _sweep_body(posx_h, posy_h, posz_h, qlo_h, qhi_h, t_h, g_h, cnt_h,
                px, py, pz, qlo_v, qhi_v, gbuf0, gbuf1, cbuf, nbuf, cq,
                rows0, rows1, gsem0, gsem1, wsem0, wsem1):
    wid = lax.axis_index("s") * 2 + lax.axis_index("c")
    base = wid * QPW
    pltpu.sync_copy(posx_h, px)
    pltpu.sync_copy(posy_h, py)
    pltpu.sync_copy(posz_h, pz)
    pltpu.sync_copy(qlo_h.at[pl.ds(base, QPW)], qlo_v)
    pltpu.sync_copy(qhi_h.at[pl.ds(base, QPW)], qhi_v)

    lanes = lax.broadcasted_iota(jnp.int32, (16,), 0)
    zeros16 = jnp.zeros((16,), jnp.int32)
    ones16 = jnp.ones((16,), jnp.int32)
    c31 = jnp.full((16,), 31, jnp.int32)
    k16 = jnp.full((16,), K, jnp.int32)
    slots = ((gbuf0, rows0, gsem0, wsem0), (gbuf1, rows1, gsem1, wsem1))
    NG = QPW // GQ
    GEB = GQ * K

    def scan_group(g, gbuf):
        goff = base + g * GQ
        qlo_g = qlo_v[pl.ds(g * GQ, 16)]
        qhi_g = qhi_v[pl.ds(g * GQ, 16)]
        qpx_g = px[pl.ds(goff, 16)]
        qpy_g = py[pl.ds(goff, 16)]
        qpz_g = pz[pl.ds(goff, 16)]
        for q in range(GQ):
            lo = qlo_g[q]
            hi = qhi_g[q]
            qx = qpx_g[q]
            qy = qpy_g[q]
            qz = qpz_g[q]
            nbuf[pl.ds(0, 16)] = zeros16
            nbuf[pl.ds(16, 16)] = zeros16
            cq[...] = zeros16
            ptr0 = (lo // 16) * 16
            nch = (hi - ptr0 + (CH - 1)) // CH

            def chunk(ci, carry):
                cnt0 = cq[...][0]

                @pl.when(cnt0 < K)
                def _():
                    cnt = cnt0
                    for u in range(CH // 16):
                        ptr = ptr0 + ci * CH + u * 16
                        ids = ptr + lanes
                        dx = px[pl.ds(ptr, 16)] - qx
                        dy = py[pl.ds(ptr, 16)] - qy
                        dz = pz[pl.ds(ptr, 16)] - qz
                        d2 = dx * dx + dy * dy + dz * dz
                        val = (d2 <= RAD2) & (ids >= lo) & (ids < hi)
                        pcs = plsc.cumsum(jnp.where(val, ones16, zeros16))
                        pos = jnp.where(val, jnp.minimum(cnt + pcs - 1, c31),
                                        c31)
                        plsc.store_scatter(nbuf, [pos], ids)
                        cnt = cnt + pcs[15]
                    cq[...] = zeros16 + cnt

                return carry

            lax.fori_loop(0, nch, chunk, 0)
            cnt_f = cq[...][0]
            nb = nbuf[pl.ds(0, 16)]
            gbuf[pl.ds(q * 16, 16)] = jnp.where(lanes < cnt_f, nb,
                                                zeros16 + nb[0])
            cbuf[...] = jnp.where(lanes == q,
                                  jnp.minimum(zeros16 + cnt_f, k16), cbuf[...])
        pltpu.sync_copy(cbuf, cnt_h.at[pl.ds(base + g * GQ, GQ)])

    def pair_body(g2, carry):
        for p in range(2):
            gbuf, rows, gsem, wsem = slots[p]
            g = g2 * 2 + p
            ebase = (base + g * GQ) * K
            eprev = ebase - 2 * GEB

            @pl.when(g2 >= 1)
            def _fire_writeback():
                pltpu.make_async_copy(g_h.at[pl.ds(0, GEB)], rows,
                                      gsem).wait()
                pltpu.async_copy(rows, g_h.at[pl.ds(eprev, GEB)], wsem)

            scan_group(g, gbuf)

            @pl.when(g2 >= 1)
            def _wait_writeback():
                pltpu.make_async_copy(rows, g_h.at[pl.ds(eprev, GEB)],
                                      wsem).wait()

            pltpu.async_copy(t_h.at[gbuf.at[pl.ds(0, GE)]],
                             rows.at[pl.ds(0, GE)], gsem)
            pltpu.async_copy(t_h.at[gbuf.at[pl.ds(GE, GE)]],
                             rows.at[pl.ds(GE, GE)], gsem)
        return carry

    lax.fori_loop(0, NG // 2, pair_body, 0)
    for p in range(2):
        gbuf, rows, gsem, wsem = slots[p]
        g = NG - 2 + p
        pltpu.make_async_copy(g_h.at[pl.ds(0, GEB)], rows, gsem).wait()
        pltpu.sync_copy(rows, g_h.at[pl.ds((base + g * GQ) * K, GEB)])


@functools.cache
def _sweep_kernel():
    return functools.partial(
        pl.kernel,
        mesh=plsc.VectorSubcoreMesh(core_axis_name="c", subcore_axis_name="s"),
        compiler_params=pltpu.CompilerParams(needs_layout_passes=False),
        out_type=[jax.ShapeDtypeStruct((EPAD, D), jnp.float32),
                  jax.ShapeDtypeStruct((NPAD,), jnp.int32)],
        scratch_types=[pltpu.VMEM((NPAD,), jnp.float32),
                       pltpu.VMEM((NPAD,), jnp.float32),
                       pltpu.VMEM((NPAD,), jnp.float32),
                       pltpu.VMEM((QPW,), jnp.int32),
                       pltpu.VMEM((QPW,), jnp.int32),
                       pltpu.VMEM((GQ * K,), jnp.int32),
                       pltpu.VMEM((GQ * K,), jnp.int32),
                       pltpu.VMEM((GQ,), jnp.int32),
                       pltpu.VMEM((32,), jnp.int32),
                       pltpu.VMEM((16,), jnp.int32),
                       pltpu.VMEM((GQ * K, D), jnp.float32),
                       pltpu.VMEM((GQ * K, D), jnp.float32),
                       pltpu.SemaphoreType.DMA,
                       pltpu.SemaphoreType.DMA,
                       pltpu.SemaphoreType.DMA,
                       pltpu.SemaphoreType.DMA],
    )(_sweep_body)


def _sweep_call(*args):
    return _sweep_kernel()(*args)


def _tv_body(x_ref, p_ref, w1xt_ref, w1pt_ref, b1_ref, t_ref, v_ref):
    v = jnp.dot(p_ref[...], w1pt_ref[...], preferred_element_type=jnp.float32,
                precision=lax.Precision.HIGHEST)
    t = jnp.dot(x_ref[...], w1xt_ref[...], preferred_element_type=jnp.float32,
                precision=lax.Precision.HIGHEST)
    v_ref[...] = v
    t_ref[...] = t + v + b1_ref[0:1, :]


def _z1_block(g_ref, v_ref):
    vexp = jnp.broadcast_to(v_ref[...].reshape(QB, 1, D), (QB, K, D))
    return _leaky(g_ref[...] - vexp.reshape(QB * K, D))


def _stats1_body(g_ref, v_ref, fm_ref, s_ref):
    z1 = _z1_block(g_ref, v_ref)
    mrow = fm_ref[0]

    @pl.when(pl.program_id(0) == 0)
    def _():
        s_ref[...] = jnp.zeros_like(s_ref)

    s_ref[0:1, :] += jnp.dot(mrow, z1, preferred_element_type=jnp.float32)
    s_ref[1:2, :] += jnp.dot(mrow, z1 * z1,
                             preferred_element_type=jnp.float32)


def _layer2_body(g_ref, v_ref, fm_ref, w2t_ref, b2_ref, mx_ref, s_ref):
    z1 = _z1_block(g_ref, v_ref)
    z2 = jnp.dot(z1, w2t_ref[...], preferred_element_type=jnp.float32,
                 precision=lax.Precision.HIGHEST)
    z2 = _leaky(z2 + b2_ref[0:1, :])
    mrow = fm_ref[0]

    @pl.when(pl.program_id(0) == 0)
    def _():
        s_ref[...] = jnp.zeros_like(s_ref)

    s_ref[0:1, :] += jnp.dot(mrow, z2, preferred_element_type=jnp.float32)
    s_ref[1:2, :] += jnp.dot(mrow, z2 * z2,
                             preferred_element_type=jnp.float32)
    mx_ref[...] = jnp.max(z2.reshape(QB, K, D), axis=1)


def _final_body(m_ref, prm_ref, o_ref):
    o_ref[...] = m_ref[...] * prm_ref[0:1, :] + prm_ref[1:2, :]


def kernel(x, pos, batch, reflectance, sf, W1, b1, g1, be1, W2, b2, g2, be2):
    f32 = jnp.float32
    pos_s = pos / sf[batch][:, None]
    ar = jnp.arange(NB, dtype=batch.dtype)
    seg_lo = jnp.searchsorted(batch, ar, side="left").astype(jnp.int32)
    seg_hi = jnp.searchsorted(batch, ar, side="right").astype(jnp.int32)
    qlo = jnp.zeros((NPAD,), jnp.int32).at[:N].set(seg_lo[batch])
    qhi = jnp.zeros((NPAD,), jnp.int32).at[:N].set(seg_hi[batch])
    posx = jnp.zeros((NPAD,), f32).at[:N].set(pos[:, 0])
    posy = jnp.zeros((NPAD,), f32).at[:N].set(pos[:, 1])
    posz = jnp.zeros((NPAD,), f32).at[:N].set(pos[:, 2])
    xp = jnp.zeros((NPAD, D), f32).at[:N].set(x)
    pp = jnp.zeros((NPAD, D), f32).at[:N, :3].set(pos_s)
    w1xt = W1[:, :D].T
    w1pt = jnp.zeros((D, D), f32).at[:3, :].set(W1[:, D:].T)
    b1r = jnp.broadcast_to(b1.reshape(1, D), (8, D))

    nb_tv = NPAD // 512
    t, v = pl.pallas_call(
        _tv_body,
        grid=(nb_tv,),
        in_specs=[pl.BlockSpec((512, D), lambda i: (i, 0)),
                  pl.BlockSpec((512, D), lambda i: (i, 0)),
                  pl.BlockSpec((D, D), lambda i: (0, 0)),
                  pl.BlockSpec((D, D), lambda i: (0, 0)),
                  pl.BlockSpec((8, D), lambda i: (0, 0))],
        out_specs=[pl.BlockSpec((512, D), lambda i: (i, 0)),
                   pl.BlockSpec((512, D), lambda i: (i, 0))],
        out_shape=[jax.ShapeDtypeStruct((NPAD, D), f32),
                   jax.ShapeDtypeStruct((NPAD, D), f32)],
    )(xp, pp, w1xt, w1pt, b1r)

    G, counts = _sweep_call(posx, posy, posz, qlo, qhi, t)

    fmask = (jnp.arange(K, dtype=jnp.int32)[None, :]
             < counts[:, None]).astype(f32)
    nblk = NPAD // QB
    fmask3 = fmask.reshape(nblk, 1, EB)
    edge_specs = [pl.BlockSpec((EB, D), lambda i: (i, 0)),
                  pl.BlockSpec((QB, D), lambda i: (i, 0)),
                  pl.BlockSpec((1, 1, EB), lambda i: (i, 0, 0))]

    s1 = pl.pallas_call(
        _stats1_body,
        grid=(nblk,),
        in_specs=edge_specs,
        out_specs=pl.BlockSpec((8, D), lambda i: (0, 0)),
        out_shape=jax.ShapeDtypeStruct((8, D), f32),
    )(G, v, fmask3)

    cnt_tot = jnp.maximum(jnp.sum(counts.astype(f32)), 1.0)
    mu1 = s1[0] / cnt_tot
    var1 = jnp.maximum(s1[1] / cnt_tot - mu1 * mu1, 0.0)
    a1 = g1 * lax.rsqrt(var1 + 1e-5)
    w2t = (W2 * a1[None, :]).T
    b2f = b2 + W2 @ (be1 - mu1 * a1)
    b2r = jnp.broadcast_to(b2f.reshape(1, D), (8, D))

    m, s2 = pl.pallas_call(
        _layer2_body,
        grid=(nblk,),
        in_specs=edge_specs + [pl.BlockSpec((D, D), lambda i: (0, 0)),
                               pl.BlockSpec((8, D), lambda i: (0, 0))],
        out_specs=[pl.BlockSpec((QB, D), lambda i: (i, 0)),
                   pl.BlockSpec((8, D), lambda i: (0, 0))],
        out_shape=[jax.ShapeDtypeStruct((NPAD, D), f32),
                   jax.ShapeDtypeStruct((8, D), f32)],
    )(G, v, fmask3, w2t, b2r)

    mu2 = s2[0] / cnt_tot
    var2 = jnp.maximum(s2[1] / cnt_tot - mu2 * mu2, 0.0)
    scale2 = g2 * lax.rsqrt(var2 + 1e-5)
    shift2 = be2 - mu2 * scale2
    prm = jnp.zeros((8, D), f32).at[0].set(scale2).at[1].set(shift2)

    out_p = pl.pallas_call(
        _final_body,
        grid=(NPAD // 512,),
        in_specs=[pl.BlockSpec((512, D), lambda i: (i, 0)),
                  pl.BlockSpec((8, D), lambda i: (0, 0))],
        out_specs=pl.BlockSpec((512, D), lambda i: (i, 0)),
        out_shape=jax.ShapeDtypeStruct((NPAD, D), f32),
    )(m, prm)

    return (out_p[:N], pos, batch, reflectance, sf)


# speedup vs baseline: 1.2720x; 1.0439x over previous
import functools

import jax
import jax.numpy as jnp
from jax import lax
from jax.experimental import pallas as pl
from jax.experimental.pallas import tpu as pltpu
from jax.experimental.pallas import tpu_sc as plsc

N = 10000
D = 128
NB = 8
K = 16
RAD2 = (0.02 * 2.1) ** 2

NW = 32
NPAD = 10240
QPW = NPAD // NW
GQ = 16
EPAD = NPAD * K
EPW = EPAD // NW
GE = 128
CH = 64
QB = 64
EB = QB * K
NEG_INF = float("-inf")


def _leaky(z):
    return jnp.where(z >= 0, z, 0.01 * z)


def _sweep_body(posx_h, posy_h, posz_h, qlo_h, qhi_h, t_h, g_h, cnt_h,
                px, py, pz, qlo_v, qhi_v, gbuf0, gbuf1, cbuf, nbuf, cq,
                rows0, rows1, gsem0, gsem1, wsem0, wsem1):
    wid = lax.axis_index("s") * 2 + lax.axis_index("c")
    base = wid * QPW
    pltpu.sync_copy(posx_h, px)
    pltpu.sync_copy(posy_h, py)
    pltpu.sync_copy(posz_h, pz)
    pltpu.sync_copy(qlo_h.at[pl.ds(base, QPW)], qlo_v)
    pltpu.sync_copy(qhi_h.at[pl.ds(base, QPW)], qhi_v)

    lanes = lax.broadcasted_iota(jnp.int32, (16,), 0)
    zeros16 = jnp.zeros((16,), jnp.int32)
    ones16 = jnp.ones((16,), jnp.int32)
    c31 = jnp.full((16,), 31, jnp.int32)
    k16 = jnp.full((16,), K, jnp.int32)
    slots = ((gbuf0, rows0, gsem0, wsem0), (gbuf1, rows1, gsem1, wsem1))
    NG = QPW // GQ
    GEB = GQ * K

    def scan_group(g, gbuf):
        goff = base + g * GQ
        qlo_g = qlo_v[pl.ds(g * GQ, 16)]
        qhi_g = qhi_v[pl.ds(g * GQ, 16)]
        qpx_g = px[pl.ds(goff, 16)]
        qpy_g = py[pl.ds(goff, 16)]
        qpz_g = pz[pl.ds(goff, 16)]
        for q in range(GQ):
            lo = qlo_g[q]
            hi = qhi_g[q]
            qx = qpx_g[q]
            qy = qpy_g[q]
            qz = qpz_g[q]
            nbuf[pl.ds(0, 16)] = zeros16
            nbuf[pl.ds(16, 16)] = zeros16
            ptr0 = (lo // 16) * 16
            nch = (hi - ptr0 + (CH - 1)) // CH

            def chunk(ci, cnt0):
                def active(cnt):
                    for u in range(CH // 16):
                        ptr = ptr0 + ci * CH + u * 16
                        ids = ptr + lanes
                        dx = px[pl.ds(ptr, 16)] - qx
                        dy = py[pl.ds(ptr, 16)] - qy
                        dz = pz[pl.ds(ptr, 16)] - qz
                        d2 = dx * dx + dy * dy + dz * dz
                        val = (d2 <= RAD2) & (ids >= lo) & (ids < hi)
                        pcs = plsc.cumsum(jnp.where(val, ones16, zeros16))
                        pos = jnp.where(val, jnp.minimum(cnt + pcs - 1, c31),
                                        c31)
                        plsc.store_scatter(nbuf, [pos], ids)
                        cnt = cnt + pcs[15]
                    return cnt

                return lax.cond(cnt0 < K, active, lambda c: c, cnt0)

            cnt_f = lax.fori_loop(0, nch, chunk, jnp.int32(0))
            nb = nbuf[pl.ds(0, 16)]
            gbuf[pl.ds(q * 16, 16)] = jnp.where(lanes < cnt_f, nb,
                                                zeros16 + nb[0])
            cbuf[...] = jnp.where(lanes == q,
                                  jnp.minimum(zeros16 + cnt_f, k16), cbuf[...])
        pltpu.sync_copy(cbuf, cnt_h.at[pl.ds(base + g * GQ, GQ)])

    def pair_body(g2, carry):
        for p in range(2):
            gbuf, rows, gsem, wsem = slots[p]
            g = g2 * 2 + p
            ebase = (base + g * GQ) * K
            eprev = ebase - 2 * GEB

            @pl.when(g2 >= 1)
            def _fire_writeback():
                pltpu.make_async_copy(g_h.at[pl.ds(0, GEB)], rows,
                                      gsem).wait()
                pltpu.async_copy(rows, g_h.at[pl.ds(eprev, GEB)], wsem)

            scan_group(g, gbuf)

            @pl.when(g2 >= 1)
            def _wait_writeback():
                pltpu.make_async_copy(rows, g_h.at[pl.ds(eprev, GEB)],
                                      wsem).wait()

            pltpu.async_copy(t_h.at[gbuf.at[pl.ds(0, GE)]],
                             rows.at[pl.ds(0, GE)], gsem)
            pltpu.async_copy(t_h.at[gbuf.at[pl.ds(GE, GE)]],
                             rows.at[pl.ds(GE, GE)], gsem)
        return carry

    lax.fori_loop(0, NG // 2, pair_body, 0)
    for p in range(2):
        gbuf, rows, gsem, wsem = slots[p]
        g = NG - 2 + p
        pltpu.make_async_copy(g_h.at[pl.ds(0, GEB)], rows, gsem).wait()
        pltpu.sync_copy(rows, g_h.at[pl.ds((base + g * GQ) * K, GEB)])


@functools.cache
def _sweep_kernel():
    return functools.partial(
        pl.kernel,
        mesh=plsc.VectorSubcoreMesh(core_axis_name="c", subcore_axis_name="s"),
        compiler_params=pltpu.CompilerParams(needs_layout_passes=False),
        out_type=[jax.ShapeDtypeStruct((EPAD, D), jnp.float32),
                  jax.ShapeDtypeStruct((NPAD,), jnp.int32)],
        scratch_types=[pltpu.VMEM((NPAD,), jnp.float32),
                       pltpu.VMEM((NPAD,), jnp.float32),
                       pltpu.VMEM((NPAD,), jnp.float32),
                       pltpu.VMEM((QPW,), jnp.int32),
                       pltpu.VMEM((QPW,), jnp.int32),
                       pltpu.VMEM((GQ * K,), jnp.int32),
                       pltpu.VMEM((GQ * K,), jnp.int32),
                       pltpu.VMEM((GQ,), jnp.int32),
                       pltpu.VMEM((32,), jnp.int32),
                       pltpu.VMEM((16,), jnp.int32),
                       pltpu.VMEM((GQ * K, D), jnp.float32),
                       pltpu.VMEM((GQ * K, D), jnp.float32),
                       pltpu.SemaphoreType.DMA,
                       pltpu.SemaphoreType.DMA,
                       pltpu.SemaphoreType.DMA,
                       pltpu.SemaphoreType.DMA],
    )(_sweep_body)


def _sweep_call(*args):
    return _sweep_kernel()(*args)


def _tv_body(x_ref, p_ref, w1xt_ref, w1pt_ref, b1_ref, t_ref, v_ref):
    v = jnp.dot(p_ref[...], w1pt_ref[...], preferred_element_type=jnp.float32,
                precision=lax.Precision.HIGHEST)
    t = jnp.dot(x_ref[...], w1xt_ref[...], preferred_element_type=jnp.float32,
                precision=lax.Precision.HIGHEST)
    v_ref[...] = v
    t_ref[...] = t + v + b1_ref[0:1, :]


def _z1_block(g_ref, v_ref):
    vexp = jnp.broadcast_to(v_ref[...].reshape(QB, 1, D), (QB, K, D))
    return _leaky(g_ref[...] - vexp.reshape(QB * K, D))


def _stats1_body(g_ref, v_ref, fm_ref, s_ref):
    z1 = _z1_block(g_ref, v_ref)
    mrow = fm_ref[0]

    @pl.when(pl.program_id(0) == 0)
    def _():
        s_ref[...] = jnp.zeros_like(s_ref)

    s_ref[0:1, :] += jnp.dot(mrow, z1, preferred_element_type=jnp.float32)
    s_ref[1:2, :] += jnp.dot(mrow, z1 * z1,
                             preferred_element_type=jnp.float32)


def _layer2_body(g_ref, v_ref, fm_ref, w2t_ref, b2_ref, mx_ref, s_ref):
    z1 = _z1_block(g_ref, v_ref)
    z2 = jnp.dot(z1, w2t_ref[...], preferred_element_type=jnp.float32,
                 precision=lax.Precision.HIGHEST)
    z2 = _leaky(z2 + b2_ref[0:1, :])
    mrow = fm_ref[0]

    @pl.when(pl.program_id(0) == 0)
    def _():
        s_ref[...] = jnp.zeros_like(s_ref)

    s_ref[0:1, :] += jnp.dot(mrow, z2, preferred_element_type=jnp.float32)
    s_ref[1:2, :] += jnp.dot(mrow, z2 * z2,
                             preferred_element_type=jnp.float32)
    mx_ref[...] = jnp.max(z2.reshape(QB, K, D), axis=1)


def _final_body(m_ref, prm_ref, o_ref):
    o_ref[...] = m_ref[...] * prm_ref[0:1, :] + prm_ref[1:2, :]


def kernel(x, pos, batch, reflectance, sf, W1, b1, g1, be1, W2, b2, g2, be2):
    f32 = jnp.float32
    pos_s = pos / sf[batch][:, None]
    ar = jnp.arange(NB, dtype=batch.dtype)
    seg_lo = jnp.searchsorted(batch, ar, side="left").astype(jnp.int32)
    seg_hi = jnp.searchsorted(batch, ar, side="right").astype(jnp.int32)
    qlo = jnp.zeros((NPAD,), jnp.int32).at[:N].set(seg_lo[batch])
    qhi = jnp.zeros((NPAD,), jnp.int32).at[:N].set(seg_hi[batch])
    posx = jnp.zeros((NPAD,), f32).at[:N].set(pos[:, 0])
    posy = jnp.zeros((NPAD,), f32).at[:N].set(pos[:, 1])
    posz = jnp.zeros((NPAD,), f32).at[:N].set(pos[:, 2])
    xp = jnp.zeros((NPAD, D), f32).at[:N].set(x)
    pp = jnp.zeros((NPAD, D), f32).at[:N, :3].set(pos_s)
    w1xt = W1[:, :D].T
    w1pt = jnp.zeros((D, D), f32).at[:3, :].set(W1[:, D:].T)
    b1r = jnp.broadcast_to(b1.reshape(1, D), (8, D))

    nb_tv = NPAD // 512
    t, v = pl.pallas_call(
        _tv_body,
        grid=(nb_tv,),
        in_specs=[pl.BlockSpec((512, D), lambda i: (i, 0)),
                  pl.BlockSpec((512, D), lambda i: (i, 0)),
                  pl.BlockSpec((D, D), lambda i: (0, 0)),
                  pl.BlockSpec((D, D), lambda i: (0, 0)),
                  pl.BlockSpec((8, D), lambda i: (0, 0))],
        out_specs=[pl.BlockSpec((512, D), lambda i: (i, 0)),
                   pl.BlockSpec((512, D), lambda i: (i, 0))],
        out_shape=[jax.ShapeDtypeStruct((NPAD, D), f32),
                   jax.ShapeDtypeStruct((NPAD, D), f32)],
    )(xp, pp, w1xt, w1pt, b1r)

    G, counts = _sweep_call(posx, posy, posz, qlo, qhi, t)

    fmask = (jnp.arange(K, dtype=jnp.int32)[None, :]
             < counts[:, None]).astype(f32)
    nblk = NPAD // QB
    fmask3 = fmask.reshape(nblk, 1, EB)
    edge_specs = [pl.BlockSpec((EB, D), lambda i: (i, 0)),
                  pl.BlockSpec((QB, D), lambda i: (i, 0)),
                  pl.BlockSpec((1, 1, EB), lambda i: (i, 0, 0))]

    s1 = pl.pallas_call(
        _stats1_body,
        grid=(nblk,),
        in_specs=edge_specs,
        out_specs=pl.BlockSpec((8, D), lambda i: (0, 0)),
        out_shape=jax.ShapeDtypeStruct((8, D), f32),
    )(G, v, fmask3)

    cnt_tot = jnp.maximum(jnp.sum(counts.astype(f32)), 1.0)
    mu1 = s1[0] / cnt_tot
    var1 = jnp.maximum(s1[1] / cnt_tot - mu1 * mu1, 0.0)
    a1 = g1 * lax.rsqrt(var1 + 1e-5)
    w2t = (W2 * a1[None, :]).T
    b2f = b2 + W2 @ (be1 - mu1 * a1)
    b2r = jnp.broadcast_to(b2f.reshape(1, D), (8, D))

    m, s2 = pl.pallas_call(
        _layer2_body,
        grid=(nblk,),
        in_specs=edge_specs + [pl.BlockSpec((D, D), lambda i: (0, 0)),
                               pl.BlockSpec((8, D), lambda i: (0, 0))],
        out_specs=[pl.BlockSpec((QB, D), lambda i: (i, 0)),
                   pl.BlockSpec((8, D), lambda i: (0, 0))],
        out_shape=[jax.ShapeDtypeStruct((NPAD, D), f32),
                   jax.ShapeDtypeStruct((8, D), f32)],
    )(G, v, fmask3, w2t, b2r)

    mu2 = s2[0] / cnt_tot
    var2 = jnp.maximum(s2[1] / cnt_tot - mu2 * mu2, 0.0)
    scale2 = g2 * lax.rsqrt(var2 + 1e-5)
    shift2 = be2 - mu2 * scale2
    prm = jnp.zeros((8, D), f32).at[0].set(scale2).at[1].set(shift2)

    out_p = pl.pallas_call(
        _final_body,
        grid=(NPAD // 512,),
        in_specs=[pl.BlockSpec((512, D), lambda i: (i, 0)),
                  pl.BlockSpec((8, D), lambda i: (0, 0))],
        out_specs=pl.BlockSpec((512, D), lambda i: (i, 0)),
        out_shape=jax.ShapeDtypeStruct((NPAD, D), f32),
    )(m, prm)

    return (out_p[:N], pos, batch, reflectance, sf)
